# baseline (speedup 1.0000x reference)
import jax
import jax.numpy as jnp
from jax import lax
from jax.experimental import pallas as pl
from jax.experimental.pallas import tpu as pltpu
from jax.experimental.pallas import tpu_sc as plsc

VOCAB = 1000000
BATCH = 16384
FIELDS = 26
EMBED = 64
NUM_WORKERS = 32
NBLK = BATCH // 128
BLK_PER_W = NBLK // NUM_WORKERS
ITEMS = FIELDS * BLK_PER_W


def _body(idx_hbm, table_hbm, out_hbm, idx_v, u_v, par_v, gbufs, obufs, gsem, ssem):
    wid = lax.axis_index("s") * 2 + lax.axis_index("c")
    w4 = wid * BLK_PER_W
    pltpu.sync_copy(idx_hbm.at[:, pl.ds(w4, BLK_PER_W), :], idx_v)

    iota = lax.iota(jnp.int32, 16)

    def prep(f, carry):
        for j in range(BLK_PER_W):
            for r0 in range(0, 128, 16):
                v = idx_v[f, j, pl.ds(r0, 16)]
                u_v[f, j, pl.ds(r0, 16)] = lax.shift_right_logical(v, 1)
                par_v[f, j, pl.ds(r0, 16)] = lax.shift_left(
                    lax.bitwise_and(v, 1), 6)
        return carry

    lax.fori_loop(0, FIELDS, prep, 0)

    def fire_gather(k, b):
        f = k // BLK_PER_W
        j = lax.rem(k, BLK_PER_W)
        pltpu.async_copy(table_hbm.at[u_v.at[f, j]], gbufs[b], gsem)

    def wait_gather(b):
        pltpu.make_async_copy(table_hbm.at[u_v.at[0, 0]], gbufs[b], gsem).wait()

    def fire_store(k, b):
        f = k // BLK_PER_W
        blk = w4 + lax.rem(k, BLK_PER_W)
        pltpu.async_copy(obufs[b], out_hbm.at[f, :, blk], ssem)

    def wait_store(b):
        pltpu.make_async_copy(obufs[b], out_hbm.at[0, :, 0], ssem).wait()

    def select(k, b):
        f = k // BLK_PER_W
        j = lax.rem(k, BLK_PER_W)
        for r0 in range(0, 128, 16):
            rows = iota + r0
            par = par_v[f, j, pl.ds(r0, 16)]

            def inner(er, carry):
                for e8 in range(8):
                    cols = par + (e8 * 8 + er)
                    vec = plsc.load_gather(gbufs[b], [rows, cols])
                    obufs[b][e8, er, pl.ds(r0, 16)] = vec
                return carry

            lax.fori_loop(0, 8, inner, 0)

    fire_gather(0, 0)

    def outer(k2, carry):
        k = k2 * 2
        for p in range(2):
            kk = k + p

            @pl.when(kk + 1 < ITEMS)
            def _():
                fire_gather(kk + 1, 1 - p)

            wait_gather(p)

            @pl.when(kk >= 2)
            def _():
                wait_store(p)

            select(kk, p)
            fire_store(kk, p)
        return carry

    lax.fori_loop(0, ITEMS // 2, outer, 0)
    wait_store(0)
    wait_store(1)


def kernel(indices, E):
    idx3 = jnp.transpose(indices).reshape(FIELDS, NBLK, 128).astype(jnp.int32)
    table2 = E.reshape(VOCAB // 2, 2 * EMBED)
    mesh = plsc.VectorSubcoreMesh(core_axis_name="c", subcore_axis_name="s")
    run = pl.kernel(
        _body,
        out_type=jax.ShapeDtypeStruct((FIELDS, 8, NBLK, 8, 128), jnp.float32),
        mesh=mesh,
        scratch_types=[
            pltpu.VMEM((FIELDS, BLK_PER_W, 128), jnp.int32),
            pltpu.VMEM((FIELDS, BLK_PER_W, 128), jnp.int32),
            pltpu.VMEM((FIELDS, BLK_PER_W, 128), jnp.int32),
            [pltpu.VMEM((128, 2 * EMBED), jnp.float32) for _ in range(2)],
            [pltpu.VMEM((8, 8, 128), jnp.float32) for _ in range(2)],
            pltpu.SemaphoreType.DMA,
            pltpu.SemaphoreType.DMA,
        ],
        compiler_params=pltpu.CompilerParams(
            use_tc_tiling_on_sc=False, needs_layout_passes=False),
    )
    p_out = run(idx3, table2)
    return p_out.transpose(2, 4, 0, 1, 3).reshape(BATCH, FIELDS, EMBED)


# speedup vs baseline: 1.5232x; 1.5232x over previous
import jax
import jax.numpy as jnp
from jax import lax
from jax.experimental import pallas as pl
from jax.experimental.pallas import tpu as pltpu
from jax.experimental.pallas import tpu_sc as plsc

VOCAB = 1000000
BATCH = 16384
FIELDS = 26
EMBED = 64
NUM_WORKERS = 32
NBLK = BATCH // 128
BLK_PER_W = NBLK // NUM_WORKERS
ITEMS = FIELDS * BLK_PER_W
GPAD = 129


def _body(idx_hbm, table_hbm, out_hbm, idx_v, gbufs, obufs, gsem, ssem):
    wid = lax.axis_index("s") * 2 + lax.axis_index("c")
    w4 = wid * BLK_PER_W
    pltpu.sync_copy(idx_hbm.at[:, pl.ds(w4, BLK_PER_W), :], idx_v)

    iota = lax.iota(jnp.int32, 16)
    e8s = [(iota + g * 16) // 8 for g in range(4)]
    ers = [lax.rem(iota + g * 16, 8) for g in range(4)]

    def fire_gather(k, b):
        f = k // BLK_PER_W
        j = lax.rem(k, BLK_PER_W)
        pltpu.async_copy(table_hbm.at[idx_v.at[f, j]], gbufs[b], gsem)

    def wait_gather(b):
        pltpu.make_async_copy(
            table_hbm.at[idx_v.at[0, 0]], gbufs[b], gsem).wait()

    def fire_store(k, b):
        f = k // BLK_PER_W
        blk = w4 + lax.rem(k, BLK_PER_W)
        pltpu.async_copy(
            obufs[b].at[:, :, pl.ds(0, 128)], out_hbm.at[f, :, blk], ssem)

    def wait_store(b):
        pltpu.make_async_copy(
            obufs[b].at[:, :, pl.ds(0, 128)], out_hbm.at[0, :, 0], ssem).wait()

    def select(b):
        def inner(c, carry):
            cs = jnp.full((16,), 0, jnp.int32) + c
            for g in range(4):
                vec = gbufs[b][c, pl.ds(g * 16, 16)]
                plsc.store_scatter(obufs[b], [e8s[g], ers[g], cs], vec)
            return carry

        lax.fori_loop(0, 128, inner, 0)

    fire_gather(0, 0)

    def outer(k2, carry):
        k = k2 * 2
        for p in range(2):
            kk = k + p

            @pl.when(kk + 1 < ITEMS)
            def _():
                fire_gather(kk + 1, 1 - p)

            wait_gather(p)

            @pl.when(kk >= 2)
            def _():
                wait_store(p)

            select(p)
            fire_store(kk, p)
        return carry

    lax.fori_loop(0, ITEMS // 2, outer, 0)
    wait_store(0)
    wait_store(1)


def kernel(indices, E):
    idx3 = jnp.transpose(indices).reshape(FIELDS, NBLK, 128).astype(jnp.int32)
    mesh = plsc.VectorSubcoreMesh(core_axis_name="c", subcore_axis_name="s")
    run = pl.kernel(
        _body,
        out_type=jax.ShapeDtypeStruct((FIELDS, 8, NBLK, 8, 128), jnp.float32),
        mesh=mesh,
        scratch_types=[
            pltpu.VMEM((FIELDS, BLK_PER_W, 128), jnp.int32),
            [pltpu.VMEM((128, EMBED), jnp.float32) for _ in range(2)],
            [pltpu.VMEM((8, 8, GPAD), jnp.float32) for _ in range(2)],
            pltpu.SemaphoreType.DMA,
            pltpu.SemaphoreType.DMA,
        ],
        compiler_params=pltpu.CompilerParams(
            use_tc_tiling_on_sc=False, needs_layout_passes=False),
    )
    p_out = run(idx3, E)
    return p_out.transpose(2, 4, 0, 1, 3).reshape(BATCH, FIELDS, EMBED)
